# initial kernel scaffold (unmeasured)
import jax
import jax.numpy as jnp
from jax import lax
from jax.experimental import pallas as pl
from jax.experimental.pallas import tpu as pltpu

N_DEV = 4
SQ = 256
D = 1024
SKV = 4096
H_PER = 8
DH = 128
SCALE = 0.08838834764831843

BF16 = jnp.bfloat16
F32 = jnp.float32


def kernel(x, Wq, Wo, K_ext, V_ext):
    def body(x_ref, wq_ref, wo_ref, k_hbm, v_hbm, out_ref,
             xall, wqb, wob, kbuf, vbuf, ps_buf, pr_buf,
             ag_send, ag_recv, ps_send, pr_recv, kv_sems):
        i = lax.axis_index("i")

        barrier_sem = pltpu.get_barrier_semaphore()
        for d in range(1, N_DEV):
            pl.semaphore_signal(
                barrier_sem, inc=1,
                device_id=((i + d) % N_DEV,),
                device_id_type=pl.DeviceIdType.MESH,
            )
        pl.semaphore_wait(barrier_sem, N_DEV - 1)

        xall[0] = x_ref[0].astype(BF16)
        ag_rdmas = []
        for d in range(1, N_DEV):
            r = pltpu.make_async_remote_copy(
                src_ref=xall.at[0],
                dst_ref=xall.at[N_DEV - d],
                send_sem=ag_send.at[d - 1],
                recv_sem=ag_recv.at[N_DEV - d],
                device_id=((i + d) % N_DEV,),
                device_id_type=pl.DeviceIdType.MESH,
            )
            r.start()
            ag_rdmas.append(r)

        wqb[...] = wq_ref[...].astype(BF16)
        wob[...] = wo_ref[...].astype(BF16)

        h0 = i * H_PER
        send_rdmas = []
        acc = None
        for d in range(N_DEV):
            b = (i + d) % N_DEV

            ck = pltpu.make_async_copy(
                k_hbm.at[b, :, pl.ds(h0, H_PER), :], kbuf, kv_sems.at[0])
            cv = pltpu.make_async_copy(
                v_hbm.at[b, :, pl.ds(h0, H_PER), :], vbuf, kv_sems.at[1])
            ck.start()
            cv.start()

            if d > 0:
                recv = pltpu.make_async_remote_copy(
                    src_ref=xall.at[0], dst_ref=xall.at[d],
                    send_sem=ag_send.at[0], recv_sem=ag_recv.at[d],
                    device_id=(i,), device_id_type=pl.DeviceIdType.MESH,
                )
                recv.wait_recv()

            q = jnp.dot(xall[d], wqb[...], preferred_element_type=F32)
            q = q.astype(BF16).reshape(SQ, H_PER, DH)

            ck.wait()
            cv.wait()

            outs = []
            for h in range(H_PER):
                qh = q[:, h, :]
                kh = kbuf[:, h, :].astype(BF16)
                s = lax.dot_general(
                    qh, kh, (((1,), (1,)), ((), ())),
                    preferred_element_type=F32) * SCALE
                m = jnp.max(s, axis=1, keepdims=True)
                p = jnp.exp(s - m)
                l = jnp.sum(p, axis=1, keepdims=True)
                vh = vbuf[:, h, :].astype(BF16)
                oh = lax.dot_general(
                    p.astype(BF16), vh, (((1,), (0,)), ((), ())),
                    preferred_element_type=F32)
                outs.append((oh / l).astype(BF16))
            o = jnp.concatenate(outs, axis=1)

            part = jnp.dot(o, wob[...], preferred_element_type=F32)

            if d == 0:
                acc = part
            else:
                ps_buf[d - 1] = part.astype(BF16)
                r = pltpu.make_async_remote_copy(
                    src_ref=ps_buf.at[d - 1],
                    dst_ref=pr_buf.at[N_DEV - d],
                    send_sem=ps_send.at[d - 1],
                    recv_sem=pr_recv.at[N_DEV - d],
                    device_id=(b,), device_id_type=pl.DeviceIdType.MESH,
                )
                r.start()
                send_rdmas.append(r)

        for s in range(1, N_DEV):
            recv = pltpu.make_async_remote_copy(
                src_ref=ps_buf.at[0], dst_ref=pr_buf.at[s],
                send_sem=ps_send.at[0], recv_sem=pr_recv.at[s],
                device_id=(i,), device_id_type=pl.DeviceIdType.MESH,
            )
            recv.wait_recv()
            acc = acc + pr_buf[s].astype(F32)

        for r in ag_rdmas:
            r.wait_send()
        for r in send_rdmas:
            r.wait_send()

        out_ref[0] = acc

    return pl.pallas_call(
        body,
        out_shape=jax.ShapeDtypeStruct((1, SQ, D), F32),
        in_specs=[
            pl.BlockSpec(memory_space=pltpu.VMEM),
            pl.BlockSpec(memory_space=pltpu.VMEM),
            pl.BlockSpec(memory_space=pltpu.VMEM),
            pl.BlockSpec(memory_space=pltpu.ANY),
            pl.BlockSpec(memory_space=pltpu.ANY),
        ],
        out_specs=pl.BlockSpec(memory_space=pltpu.VMEM),
        scratch_shapes=[
            pltpu.VMEM((N_DEV, SQ, D), BF16),
            pltpu.VMEM((D, D), BF16),
            pltpu.VMEM((D, D), BF16),
            pltpu.VMEM((SKV, H_PER, DH), F32),
            pltpu.VMEM((SKV, H_PER, DH), F32),
            pltpu.VMEM((N_DEV - 1, SQ, D), BF16),
            pltpu.VMEM((N_DEV, SQ, D), BF16),
            pltpu.SemaphoreType.DMA((N_DEV - 1,)),
            pltpu.SemaphoreType.DMA((N_DEV,)),
            pltpu.SemaphoreType.DMA((N_DEV - 1,)),
            pltpu.SemaphoreType.DMA((N_DEV,)),
            pltpu.SemaphoreType.DMA((2,)),
        ],
        compiler_params=pltpu.CompilerParams(collective_id=0),
    )(x, Wq, Wo, K_ext, V_ext)


# baseline (device time: 103750 ns/iter reference)
import os

import jax
import jax.numpy as jnp
from jax import lax
from jax.experimental import pallas as pl
from jax.experimental.pallas import tpu as pltpu

N_DEV = 4
SQ = 256
D = 1024
SKV = 4096
H_PER = 8
DH = 128
SCALE = 0.08838834764831843

BF16 = jnp.bfloat16
F32 = jnp.float32


def kernel(x, Wq, Wo, K_ext, V_ext):
    def body(x_ref, wq_ref, wo_ref, k_hbm, v_hbm, out_ref,
             xall, wqb, wob, kbuf, vbuf, obuf, ps_buf, pr_buf,
             ag_send, ag_recv, ps_send, pr_recv, kv_sems):
        i = lax.axis_index("i")

        barrier_sem = pltpu.get_barrier_semaphore()
        for d in range(1, N_DEV):
            pl.semaphore_signal(
                barrier_sem, inc=1,
                device_id=((i + d) % N_DEV,),
                device_id_type=pl.DeviceIdType.MESH,
            )
        pl.semaphore_wait(barrier_sem, N_DEV - 1)

        xall[0] = x_ref[0].astype(BF16)
        ag_rdmas = []
        for d in range(1, N_DEV):
            r = pltpu.make_async_remote_copy(
                src_ref=xall.at[0],
                dst_ref=xall.at[N_DEV - d],
                send_sem=ag_send.at[d - 1],
                recv_sem=ag_recv.at[N_DEV - d],
                device_id=((i + d) % N_DEV,),
                device_id_type=pl.DeviceIdType.MESH,
            )
            r.start()
            ag_rdmas.append(r)

        wqb[...] = wq_ref[...].astype(BF16)
        wob[...] = wo_ref[...].astype(BF16)

        h0 = i * H_PER

        PF = 4

        def start_kv(j):
            d, h = divmod(j, H_PER)
            b = (i + d) % N_DEV
            slot = j % PF
            ck = pltpu.make_async_copy(
                k_hbm.at[b, :, h0 + h, :], kbuf.at[slot],
                kv_sems.at[slot, 0])
            cv = pltpu.make_async_copy(
                v_hbm.at[b, :, h0 + h, :], vbuf.at[slot],
                kv_sems.at[slot, 1])
            ck.start()
            cv.start()
            return ck, cv

        kv_inflight = {j: start_kv(j) for j in range(PF)}

        send_rdmas = []
        acc = None
        for d in range(N_DEV):
            b = (i + d) % N_DEV

            if d > 0:
                recv = pltpu.make_async_remote_copy(
                    src_ref=xall.at[0], dst_ref=xall.at[d],
                    send_sem=ag_send.at[0], recv_sem=ag_recv.at[d],
                    device_id=(i,), device_id_type=pl.DeviceIdType.MESH,
                )
                recv.wait_recv()

            q = jnp.dot(xall[d], wqb[...], preferred_element_type=F32)
            q = q.astype(BF16)

            for h in range(H_PER):
                j = d * H_PER + h
                slot = j % PF
                ck, cv = kv_inflight.pop(j)
                qh = q[:, h * DH:(h + 1) * DH]
                ck.wait()
                kh = kbuf[slot].astype(BF16)
                s = lax.dot_general(
                    qh, kh, (((1,), (1,)), ((), ())),
                    preferred_element_type=F32) * SCALE
                m = jnp.max(s, axis=1, keepdims=True)
                p = jnp.exp(s - m)
                l = jnp.sum(p, axis=1, keepdims=True)
                cv.wait()
                vh = vbuf[slot].astype(BF16)
                oh = lax.dot_general(
                    p.astype(BF16), vh, (((1,), (0,)), ((), ())),
                    preferred_element_type=F32)
                obuf[:, h * DH:(h + 1) * DH] = (oh / l).astype(BF16)
                if j + PF < N_DEV * H_PER:
                    kv_inflight[j + PF] = start_kv(j + PF)

            part = jnp.dot(obuf[...], wob[...], preferred_element_type=F32)

            if d == 0:
                acc = part
            else:
                ps_buf[d - 1] = part.astype(BF16)
                r = pltpu.make_async_remote_copy(
                    src_ref=ps_buf.at[d - 1],
                    dst_ref=pr_buf.at[N_DEV - d],
                    send_sem=ps_send.at[d - 1],
                    recv_sem=pr_recv.at[N_DEV - d],
                    device_id=(b,), device_id_type=pl.DeviceIdType.MESH,
                )
                r.start()
                send_rdmas.append(r)

        for s in range(1, N_DEV):
            recv = pltpu.make_async_remote_copy(
                src_ref=ps_buf.at[0], dst_ref=pr_buf.at[s],
                send_sem=ps_send.at[0], recv_sem=pr_recv.at[s],
                device_id=(i,), device_id_type=pl.DeviceIdType.MESH,
            )
            recv.wait_recv()
            acc = acc + pr_buf[s].astype(F32)

        for r in ag_rdmas:
            r.wait_send()
        for r in send_rdmas:
            r.wait_send()

        out_ref[0] = acc

    return pl.pallas_call(
        body,
        out_shape=jax.ShapeDtypeStruct((1, SQ, D), F32),
        in_specs=[
            pl.BlockSpec(memory_space=pltpu.VMEM),
            pl.BlockSpec(memory_space=pltpu.VMEM),
            pl.BlockSpec(memory_space=pltpu.VMEM),
            pl.BlockSpec(memory_space=pl.ANY),
            pl.BlockSpec(memory_space=pl.ANY),
        ],
        out_specs=pl.BlockSpec(memory_space=pltpu.VMEM),
        scratch_shapes=[
            pltpu.VMEM((N_DEV, SQ, D), BF16),
            pltpu.VMEM((D, D), BF16),
            pltpu.VMEM((D, D), BF16),
            pltpu.VMEM((4, SKV, DH), F32),
            pltpu.VMEM((4, SKV, DH), F32),
            pltpu.VMEM((SQ, D), BF16),
            pltpu.VMEM((N_DEV - 1, SQ, D), BF16),
            pltpu.VMEM((N_DEV, SQ, D), BF16),
            pltpu.SemaphoreType.DMA((N_DEV - 1,)),
            pltpu.SemaphoreType.DMA((N_DEV,)),
            pltpu.SemaphoreType.DMA((N_DEV - 1,)),
            pltpu.SemaphoreType.DMA((N_DEV,)),
            pltpu.SemaphoreType.DMA((4, 2)),
        ],
        compiler_params=pltpu.CompilerParams(
            collective_id=0,
            vmem_limit_bytes=100 * 1024 * 1024,
        ),
        interpret=(
            pltpu.InterpretParams()
            if os.environ.get("PL_INTERPRET") == "1"
            else False
        ),
    )(x, Wq, Wo, K_ext, V_ext)
